# Initial kernel scaffold; baseline (speedup 1.0000x reference)
import jax
import jax.numpy as jnp
from jax.experimental import pallas as pl


def kernel(x, edge_index, batch, Wl1, bl1, Wr1, Wl2, bl2, Wr2, Wl3, bl3, Wr3, Wl4, bl4, Wr4, fcW, fcb):
    raise NotImplementedError("write your pallas kernel here")


# speedup vs baseline: 1.0024x; 1.0024x over previous
import jax
import jax.numpy as jnp
from jax.experimental import pallas as pl

N_NODES = 50000
N_GRAPHS = 64


def _sage(x, src, dst, Wl, bl, Wr):
    msg = jnp.take(x, src, axis=0)
    agg_sum = jax.ops.segment_sum(msg, dst, num_segments=N_NODES)
    deg = jax.ops.segment_sum(jnp.ones(src.shape, dtype=x.dtype), dst, num_segments=N_NODES)
    agg = agg_sum / jnp.clip(deg, 1.0, None)[:, None]
    return agg @ Wl + bl + x @ Wr


def _head_body(pooled_ref, fcW_ref, fcb_ref, out_ref):
    out = jnp.dot(pooled_ref[...], fcW_ref[...], preferred_element_type=jnp.float32) + fcb_ref[...]
    norm = jnp.sqrt(jnp.sum(out * out, axis=1, keepdims=True))
    out_ref[...] = out / jnp.maximum(norm, 1e-12)


def kernel(x, edge_index, batch, Wl1, bl1, Wr1, Wl2, bl2, Wr2, Wl3, bl3, Wr3, Wl4, bl4, Wr4, fcW, fcb):
    src = edge_index[0]
    dst = edge_index[1]
    h = jax.nn.relu(_sage(x, src, dst, Wl1, bl1, Wr1))
    h = jax.nn.relu(_sage(h, src, dst, Wl2, bl2, Wr2))
    h = jax.nn.relu(_sage(h, src, dst, Wl3, bl3, Wr3))
    h = jax.nn.relu(_sage(h, src, dst, Wl4, bl4, Wr4))
    sums = jax.ops.segment_sum(h, batch, num_segments=N_GRAPHS)
    counts = jax.ops.segment_sum(jnp.ones((h.shape[0],), dtype=h.dtype), batch, num_segments=N_GRAPHS)
    pooled = sums / jnp.clip(counts, 1.0, None)[:, None]
    out = pl.pallas_call(
        _head_body,
        out_shape=jax.ShapeDtypeStruct((N_GRAPHS, fcW.shape[1]), jnp.float32),
    )(pooled, fcW, fcb.reshape(1, -1))
    return out


# speedup vs baseline: 8.8082x; 8.7872x over previous
import jax
import jax.numpy as jnp
from jax import lax
from jax.experimental import pallas as pl
from jax.experimental.pallas import tpu as pltpu
from jax.experimental.pallas import tpu_sc as plsc

N = 50000
E = 800000
G = 64
K = 125
NCHUNK = E // K
NB = 4
RB = 2000
GRID = N // RB
NS = 16
STRIPE = 3128
LAST_STRIPE = N - 15 * STRIPE


def _mk_mesh():
    return plsc.VectorSubcoreMesh(
        core_axis_name="c", subcore_axis_name="s", num_cores=2, num_subcores=NS)


def _make_agg(C, edge_split, table_3d):
    cpt = (NCHUNK // 32) if edge_split else (NCHUNK // NS)
    outer = cpt // NB

    def body(table, src2, dst2, zeros, out, src_sv, dst_sv, rows, acc, sem):
        s = lax.axis_index("s")
        c = lax.axis_index("c")

        def core_half(cc):
            tref = table.at[cc] if table_3d else table

            @pl.when(s < 15)
            def _():
                pltpu.sync_copy(zeros, acc.at[pl.ds(s * STRIPE, STRIPE), :])

            @pl.when(s == 15)
            def _():
                pltpu.sync_copy(zeros.at[pl.ds(0, LAST_STRIPE), :],
                                acc.at[pl.ds(15 * STRIPE, LAST_STRIPE), :])

            plsc.subcore_barrier()
            chunk0 = ((cc * NS + s) * cpt) if edge_split else (s * cpt)

            def obody(o, carry):
                base = chunk0 + o * NB
                pltpu.sync_copy(src2.at[pl.ds(base, NB)], src_sv)
                pltpu.sync_copy(dst2.at[pl.ds(base, NB)], dst_sv)
                handles = [
                    pltpu.async_copy(tref.at[src_sv.at[b]], rows.at[b], sem)
                    for b in range(NB)
                ]
                for h in handles:
                    h.wait()
                for b in range(NB):
                    pltpu.sync_copy(rows.at[b], acc.at[dst_sv.at[b]], add=True)
                return carry

            lax.fori_loop(0, outer, obody, 0)
            plsc.subcore_barrier()

            @pl.when(s < 15)
            def _():
                pltpu.sync_copy(acc.at[pl.ds(s * STRIPE, STRIPE), :],
                                out.at[cc, pl.ds(s * STRIPE, STRIPE), :])

            @pl.when(s == 15)
            def _():
                pltpu.sync_copy(acc.at[pl.ds(15 * STRIPE, LAST_STRIPE), :],
                                out.at[cc, pl.ds(15 * STRIPE, LAST_STRIPE), :])

        @pl.when(c == 0)
        def _():
            core_half(0)

        @pl.when(c == 1)
        def _():
            core_half(1)

    return pl.kernel(
        body,
        out_type=jax.ShapeDtypeStruct((2, N, C), jnp.float32),
        mesh=_mk_mesh(),
        compiler_params=pltpu.CompilerParams(use_tc_tiling_on_sc=False),
        scratch_types=[
            pltpu.VMEM((NB, K), jnp.int32),
            pltpu.VMEM((NB, K), jnp.int32),
            pltpu.VMEM((NB, K, C), jnp.float32),
            pltpu.VMEM_SHARED((N, C), jnp.float32),
            pltpu.SemaphoreType.DMA,
        ],
    )


def _pool_body(hp3, batch3, zerosp, out, rowbuf, bidx, accp):
    s = lax.axis_index("s")
    c = lax.axis_index("c")
    wid = c * NS + s

    @pl.when(s < 8)
    def _():
        pltpu.sync_copy(zerosp, accp.at[pl.ds(s * 8, 8), :])

    plsc.subcore_barrier()

    def jbody(j, carry):
        chunk = wid * 13 + j

        @pl.when(chunk < 400)
        def _():
            pltpu.sync_copy(batch3.at[chunk], bidx)
            pltpu.sync_copy(hp3.at[chunk], rowbuf)
            pltpu.sync_copy(rowbuf, accp.at[bidx.at[0]], add=True)

        return carry

    lax.fori_loop(0, 13, jbody, 0)
    plsc.subcore_barrier()

    @pl.when(s == 0)
    def _():
        @pl.when(c == 0)
        def _():
            pltpu.sync_copy(accp, out.at[0])

        @pl.when(c == 1)
        def _():
            pltpu.sync_copy(accp, out.at[1])


def _make_pool():
    return pl.kernel(
        _pool_body,
        out_type=jax.ShapeDtypeStruct((2, G, 80), jnp.float32),
        mesh=_mk_mesh(),
        compiler_params=pltpu.CompilerParams(use_tc_tiling_on_sc=False),
        scratch_types=[
            pltpu.VMEM((K, 80), jnp.float32),
            pltpu.VMEM((1, K), jnp.int32),
            pltpu.VMEM_SHARED((G, 80), jnp.float32),
        ],
    )


def _relu(v):
    return jnp.maximum(v, 0.0)


def _tc_layer1(s1_ref, x_ref, wl1, wr1, b1, wl2, wr2, b2, p2_ref, q2_ref, dinv_ref):
    s1 = s1_ref[0] + s1_ref[1]
    invd = 1.0 / jnp.maximum(s1[:, 3:4], 1.0)
    agg = s1[:, 0:3] * invd
    h = _relu(jnp.dot(agg, wl1[...], preferred_element_type=jnp.float32)
              + jnp.dot(x_ref[...], wr1[...], preferred_element_type=jnp.float32)
              + b1[...])
    p = jnp.dot(h, wl2[...], preferred_element_type=jnp.float32)
    p2_ref[0] = p[:, 0:32]
    p2_ref[1] = p[:, 32:64]
    q2_ref[...] = jnp.dot(h, wr2[...], preferred_element_type=jnp.float32) + b2[...]
    dinv_ref[...] = jnp.broadcast_to(invd, (invd.shape[0], 8))


def _tc_mid(t_ref, q_ref, dinv_ref, wl, wr, b, pn_ref, qn_ref):
    t = jnp.concatenate([t_ref[0], t_ref[1]], axis=1)
    h = _relu(t * dinv_ref[:, 0:1] + q_ref[...])
    p = jnp.dot(h, wl[...], preferred_element_type=jnp.float32)
    pn_ref[0] = p[:, 0:32]
    pn_ref[1] = p[:, 32:64]
    qn_ref[...] = jnp.dot(h, wr[...], preferred_element_type=jnp.float32) + b[...]


def _tc_last(t_ref, q_ref, dinv_ref, hp_ref):
    t = jnp.concatenate([t_ref[0], t_ref[1]], axis=1)
    h = _relu(t * dinv_ref[:, 0:1] + q_ref[...])
    rows = h.shape[0]
    hp_ref[...] = jnp.concatenate(
        [h, jnp.ones((rows, 1), jnp.float32), jnp.zeros((rows, 15), jnp.float32)],
        axis=1)


def _tc_head(pool_ref, fcw, fcb, out_ref):
    sm = pool_ref[0] + pool_ref[1]
    mean = sm[:, 0:64] / jnp.maximum(sm[:, 64:65], 1.0)
    o = jnp.dot(mean, fcw[...], preferred_element_type=jnp.float32) + fcb[...]
    nrm = jnp.sqrt(jnp.sum(o * o, axis=1, keepdims=True))
    out_ref[...] = o / jnp.maximum(nrm, 1e-12)


def _full(shape):
    return pl.BlockSpec(shape, lambda *args: tuple(0 for _ in shape))


def kernel(x, edge_index, batch, Wl1, bl1, Wr1, Wl2, bl2, Wr2, Wl3, bl3, Wr3,
           Wl4, bl4, Wr4, fcW, fcb):
    src2 = edge_index[0].reshape(NCHUNK, K)
    dst2 = edge_index[1].reshape(NCHUNK, K)
    batch3 = batch.reshape(400, 1, K)
    xpad = jnp.concatenate(
        [x, jnp.ones((N, 1), jnp.float32), jnp.zeros((N, 12), jnp.float32)], axis=1)
    zeros16 = jnp.zeros((STRIPE, 16), jnp.float32)
    zeros32 = jnp.zeros((STRIPE, 32), jnp.float32)
    zerosp = jnp.zeros((8, 80), jnp.float32)

    s1 = _make_agg(16, edge_split=True, table_3d=False)(xpad, src2, dst2, zeros16)

    p2, q2, dinv = pl.pallas_call(
        _tc_layer1,
        grid=(GRID,),
        in_specs=[
            pl.BlockSpec((2, RB, 16), lambda i: (0, i, 0)),
            pl.BlockSpec((RB, 3), lambda i: (i, 0)),
            _full((3, 64)), _full((3, 64)), _full((1, 64)),
            _full((64, 64)), _full((64, 64)), _full((1, 64)),
        ],
        out_specs=[
            pl.BlockSpec((2, RB, 32), lambda i: (0, i, 0)),
            pl.BlockSpec((RB, 64), lambda i: (i, 0)),
            pl.BlockSpec((RB, 8), lambda i: (i, 0)),
        ],
        out_shape=[
            jax.ShapeDtypeStruct((2, N, 32), jnp.float32),
            jax.ShapeDtypeStruct((N, 64), jnp.float32),
            jax.ShapeDtypeStruct((N, 8), jnp.float32),
        ],
    )(s1, x, Wl1, Wr1, bl1.reshape(1, 64), Wl2, Wr2, bl2.reshape(1, 64))

    agg32 = _make_agg(32, edge_split=False, table_3d=True)

    def mid(t, q, wl, wr, b):
        return pl.pallas_call(
            _tc_mid,
            grid=(GRID,),
            in_specs=[
                pl.BlockSpec((2, RB, 32), lambda i: (0, i, 0)),
                pl.BlockSpec((RB, 64), lambda i: (i, 0)),
                pl.BlockSpec((RB, 8), lambda i: (i, 0)),
                _full((64, 64)), _full((64, 64)), _full((1, 64)),
            ],
            out_specs=[
                pl.BlockSpec((2, RB, 32), lambda i: (0, i, 0)),
                pl.BlockSpec((RB, 64), lambda i: (i, 0)),
            ],
            out_shape=[
                jax.ShapeDtypeStruct((2, N, 32), jnp.float32),
                jax.ShapeDtypeStruct((N, 64), jnp.float32),
            ],
        )(t, q, dinv, wl, wr, b.reshape(1, 64))

    t2 = agg32(p2, src2, dst2, zeros32)
    p3, q3 = mid(t2, q2, Wl3, Wr3, bl3)
    t3 = agg32(p3, src2, dst2, zeros32)
    p4, q4 = mid(t3, q3, Wl4, Wr4, bl4)
    t4 = agg32(p4, src2, dst2, zeros32)

    hp = pl.pallas_call(
        _tc_last,
        grid=(GRID,),
        in_specs=[
            pl.BlockSpec((2, RB, 32), lambda i: (0, i, 0)),
            pl.BlockSpec((RB, 64), lambda i: (i, 0)),
            pl.BlockSpec((RB, 8), lambda i: (i, 0)),
        ],
        out_specs=pl.BlockSpec((RB, 80), lambda i: (i, 0)),
        out_shape=jax.ShapeDtypeStruct((N, 80), jnp.float32),
    )(t4, q4, dinv)

    pool = _make_pool()(hp.reshape(400, K, 80), batch3, zerosp)

    out = pl.pallas_call(
        _tc_head,
        in_specs=[_full((2, G, 80)), _full((64, 128)), _full((1, 128))],
        out_specs=_full((G, 128)),
        out_shape=jax.ShapeDtypeStruct((G, 128), jnp.float32),
    )(pool, fcW, fcb.reshape(1, 128))
    return out


# speedup vs baseline: 10.6863x; 1.2132x over previous
import jax
import jax.numpy as jnp
from jax import lax
from jax.experimental import pallas as pl
from jax.experimental.pallas import tpu as pltpu
from jax.experimental.pallas import tpu_sc as plsc

N = 50000
E = 800000
G = 64
K = 125
NCHUNK = E // K
NB = 4
RB = 2000
GRID = N // RB
NS = 16
STRIPE = 3128
LAST_STRIPE = N - 15 * STRIPE


def _mk_mesh():
    return plsc.VectorSubcoreMesh(
        core_axis_name="c", subcore_axis_name="s", num_cores=2, num_subcores=NS)


def _make_agg(C, edge_split, table_3d):
    cpt = (NCHUNK // 32) if edge_split else (NCHUNK // NS)
    SB = 8 if edge_split else 16
    outer = cpt // SB
    D = 4

    def body(table, src2, dst2, zeros, out, src_sv, dst_sv, rows, acc, gsem, ssem):
        s = lax.axis_index("s")
        c = lax.axis_index("c")

        def core_half(cc):
            tref = table.at[cc] if table_3d else table

            @pl.when(s < 15)
            def _():
                pltpu.sync_copy(zeros, acc.at[pl.ds(s * STRIPE, STRIPE), :])

            @pl.when(s == 15)
            def _():
                pltpu.sync_copy(zeros.at[pl.ds(0, LAST_STRIPE), :],
                                acc.at[pl.ds(15 * STRIPE, LAST_STRIPE), :])

            plsc.subcore_barrier()
            chunk0 = ((cc * NS + s) * cpt) if edge_split else (s * cpt)

            def obody(o, carry):
                base = chunk0 + o * SB
                pltpu.sync_copy(src2.at[pl.ds(base, SB)], src_sv)
                pltpu.sync_copy(dst2.at[pl.ds(base, SB)], dst_sv)
                hg = [None] * SB
                hs = [None] * SB
                for j in range(SB):
                    par = j % D
                    if j >= D:
                        hs[j - D].wait()
                    hg[j] = pltpu.async_copy(
                        tref.at[src_sv.at[j]], rows.at[par], gsem)
                    if j >= 1:
                        hg[j - 1].wait()
                        hs[j - 1] = pltpu.async_copy(
                            rows.at[(j - 1) % D], acc.at[dst_sv.at[j - 1]],
                            ssem, add=True)
                hg[SB - 1].wait()
                hs[SB - 1] = pltpu.async_copy(
                    rows.at[(SB - 1) % D], acc.at[dst_sv.at[SB - 1]],
                    ssem, add=True)
                for j in range(SB - D, SB):
                    hs[j].wait()
                return carry

            lax.fori_loop(0, outer, obody, 0)
            plsc.subcore_barrier()

            @pl.when(s < 15)
            def _():
                pltpu.sync_copy(acc.at[pl.ds(s * STRIPE, STRIPE), :],
                                out.at[cc, pl.ds(s * STRIPE, STRIPE), :])

            @pl.when(s == 15)
            def _():
                pltpu.sync_copy(acc.at[pl.ds(15 * STRIPE, LAST_STRIPE), :],
                                out.at[cc, pl.ds(15 * STRIPE, LAST_STRIPE), :])

        @pl.when(c == 0)
        def _():
            core_half(0)

        @pl.when(c == 1)
        def _():
            core_half(1)

    return pl.kernel(
        body,
        out_type=jax.ShapeDtypeStruct((2, N, C), jnp.float32),
        mesh=_mk_mesh(),
        compiler_params=pltpu.CompilerParams(use_tc_tiling_on_sc=False),
        scratch_types=[
            pltpu.VMEM((SB, K), jnp.int32),
            pltpu.VMEM((SB, K), jnp.int32),
            pltpu.VMEM((D, K, C), jnp.float32),
            pltpu.VMEM_SHARED((N, C), jnp.float32),
            pltpu.SemaphoreType.DMA,
            pltpu.SemaphoreType.DMA,
        ],
    )


def _pool_body(hp3, batch3, zerosp, out, rowbuf, bidx, accp):
    s = lax.axis_index("s")
    c = lax.axis_index("c")
    wid = c * NS + s

    @pl.when(s < 8)
    def _():
        pltpu.sync_copy(zerosp, accp.at[pl.ds(s * 8, 8), :])

    plsc.subcore_barrier()

    def jbody(j, carry):
        chunk = wid * 13 + j

        @pl.when(chunk < 400)
        def _():
            pltpu.sync_copy(batch3.at[chunk], bidx)
            pltpu.sync_copy(hp3.at[chunk], rowbuf)
            pltpu.sync_copy(rowbuf, accp.at[bidx.at[0]], add=True)

        return carry

    lax.fori_loop(0, 13, jbody, 0)
    plsc.subcore_barrier()

    @pl.when(s == 0)
    def _():
        @pl.when(c == 0)
        def _():
            pltpu.sync_copy(accp, out.at[0])

        @pl.when(c == 1)
        def _():
            pltpu.sync_copy(accp, out.at[1])


def _make_pool():
    return pl.kernel(
        _pool_body,
        out_type=jax.ShapeDtypeStruct((2, G, 80), jnp.float32),
        mesh=_mk_mesh(),
        compiler_params=pltpu.CompilerParams(use_tc_tiling_on_sc=False),
        scratch_types=[
            pltpu.VMEM((K, 80), jnp.float32),
            pltpu.VMEM((1, K), jnp.int32),
            pltpu.VMEM_SHARED((G, 80), jnp.float32),
        ],
    )


def _relu(v):
    return jnp.maximum(v, 0.0)


def _tc_layer1(s1_ref, x_ref, wl1, wr1, b1, wl2, wr2, b2, p2_ref, q2_ref, dinv_ref):
    s1 = s1_ref[0] + s1_ref[1]
    invd = 1.0 / jnp.maximum(s1[:, 3:4], 1.0)
    agg = s1[:, 0:3] * invd
    h = _relu(jnp.dot(agg, wl1[...], preferred_element_type=jnp.float32)
              + jnp.dot(x_ref[...], wr1[...], preferred_element_type=jnp.float32)
              + b1[...])
    p = jnp.dot(h, wl2[...], preferred_element_type=jnp.float32)
    p2_ref[0] = p[:, 0:32]
    p2_ref[1] = p[:, 32:64]
    q2_ref[...] = jnp.dot(h, wr2[...], preferred_element_type=jnp.float32) + b2[...]
    dinv_ref[...] = jnp.broadcast_to(invd, (invd.shape[0], 8))


def _tc_mid(t_ref, q_ref, dinv_ref, wl, wr, b, pn_ref, qn_ref):
    t = jnp.concatenate([t_ref[0], t_ref[1]], axis=1)
    h = _relu(t * dinv_ref[:, 0:1] + q_ref[...])
    p = jnp.dot(h, wl[...], preferred_element_type=jnp.float32)
    pn_ref[0] = p[:, 0:32]
    pn_ref[1] = p[:, 32:64]
    qn_ref[...] = jnp.dot(h, wr[...], preferred_element_type=jnp.float32) + b[...]


def _tc_last(t_ref, q_ref, dinv_ref, hp_ref):
    t = jnp.concatenate([t_ref[0], t_ref[1]], axis=1)
    h = _relu(t * dinv_ref[:, 0:1] + q_ref[...])
    rows = h.shape[0]
    hp_ref[...] = jnp.concatenate(
        [h, jnp.ones((rows, 1), jnp.float32), jnp.zeros((rows, 15), jnp.float32)],
        axis=1)


def _tc_head(pool_ref, fcw, fcb, out_ref):
    sm = pool_ref[0] + pool_ref[1]
    mean = sm[:, 0:64] / jnp.maximum(sm[:, 64:65], 1.0)
    o = jnp.dot(mean, fcw[...], preferred_element_type=jnp.float32) + fcb[...]
    nrm = jnp.sqrt(jnp.sum(o * o, axis=1, keepdims=True))
    out_ref[...] = o / jnp.maximum(nrm, 1e-12)


def _full(shape):
    return pl.BlockSpec(shape, lambda *args: tuple(0 for _ in shape))


def kernel(x, edge_index, batch, Wl1, bl1, Wr1, Wl2, bl2, Wr2, Wl3, bl3, Wr3,
           Wl4, bl4, Wr4, fcW, fcb):
    src2 = edge_index[0].reshape(NCHUNK, K)
    dst2 = edge_index[1].reshape(NCHUNK, K)
    batch3 = batch.reshape(400, 1, K)
    xpad = jnp.concatenate(
        [x, jnp.ones((N, 1), jnp.float32), jnp.zeros((N, 12), jnp.float32)], axis=1)
    zeros16 = jnp.zeros((STRIPE, 16), jnp.float32)
    zeros32 = jnp.zeros((STRIPE, 32), jnp.float32)
    zerosp = jnp.zeros((8, 80), jnp.float32)

    s1 = _make_agg(16, edge_split=True, table_3d=False)(xpad, src2, dst2, zeros16)

    p2, q2, dinv = pl.pallas_call(
        _tc_layer1,
        grid=(GRID,),
        in_specs=[
            pl.BlockSpec((2, RB, 16), lambda i: (0, i, 0)),
            pl.BlockSpec((RB, 3), lambda i: (i, 0)),
            _full((3, 64)), _full((3, 64)), _full((1, 64)),
            _full((64, 64)), _full((64, 64)), _full((1, 64)),
        ],
        out_specs=[
            pl.BlockSpec((2, RB, 32), lambda i: (0, i, 0)),
            pl.BlockSpec((RB, 64), lambda i: (i, 0)),
            pl.BlockSpec((RB, 8), lambda i: (i, 0)),
        ],
        out_shape=[
            jax.ShapeDtypeStruct((2, N, 32), jnp.float32),
            jax.ShapeDtypeStruct((N, 64), jnp.float32),
            jax.ShapeDtypeStruct((N, 8), jnp.float32),
        ],
    )(s1, x, Wl1, Wr1, bl1.reshape(1, 64), Wl2, Wr2, bl2.reshape(1, 64))

    agg32 = _make_agg(32, edge_split=False, table_3d=True)

    def mid(t, q, wl, wr, b):
        return pl.pallas_call(
            _tc_mid,
            grid=(GRID,),
            in_specs=[
                pl.BlockSpec((2, RB, 32), lambda i: (0, i, 0)),
                pl.BlockSpec((RB, 64), lambda i: (i, 0)),
                pl.BlockSpec((RB, 8), lambda i: (i, 0)),
                _full((64, 64)), _full((64, 64)), _full((1, 64)),
            ],
            out_specs=[
                pl.BlockSpec((2, RB, 32), lambda i: (0, i, 0)),
                pl.BlockSpec((RB, 64), lambda i: (i, 0)),
            ],
            out_shape=[
                jax.ShapeDtypeStruct((2, N, 32), jnp.float32),
                jax.ShapeDtypeStruct((N, 64), jnp.float32),
            ],
        )(t, q, dinv, wl, wr, b.reshape(1, 64))

    t2 = agg32(p2, src2, dst2, zeros32)
    p3, q3 = mid(t2, q2, Wl3, Wr3, bl3)
    t3 = agg32(p3, src2, dst2, zeros32)
    p4, q4 = mid(t3, q3, Wl4, Wr4, bl4)
    t4 = agg32(p4, src2, dst2, zeros32)

    hp = pl.pallas_call(
        _tc_last,
        grid=(GRID,),
        in_specs=[
            pl.BlockSpec((2, RB, 32), lambda i: (0, i, 0)),
            pl.BlockSpec((RB, 64), lambda i: (i, 0)),
            pl.BlockSpec((RB, 8), lambda i: (i, 0)),
        ],
        out_specs=pl.BlockSpec((RB, 80), lambda i: (i, 0)),
        out_shape=jax.ShapeDtypeStruct((N, 80), jnp.float32),
    )(t4, q4, dinv)

    pool = _make_pool()(hp.reshape(400, K, 80), batch3, zerosp)

    out = pl.pallas_call(
        _tc_head,
        in_specs=[_full((2, G, 80)), _full((64, 128)), _full((1, 128))],
        out_specs=_full((G, 128)),
        out_shape=jax.ShapeDtypeStruct((G, 128), jnp.float32),
    )(pool, fcW, fcb.reshape(1, 128))
    return out


# speedup vs baseline: 11.9670x; 1.1198x over previous
import jax
import jax.numpy as jnp
from jax import lax
from jax.experimental import pallas as pl
from jax.experimental.pallas import tpu as pltpu
from jax.experimental.pallas import tpu_sc as plsc

N = 50000
E = 800000
G = 64
K = 125
NCHUNK = E // K
RB = 2000
GRID = N // RB
NS = 16
STRIPE = 3128
LAST_STRIPE = N - 15 * STRIPE


def _mk_mesh():
    return plsc.VectorSubcoreMesh(
        core_axis_name="c", subcore_axis_name="s", num_cores=2, num_subcores=NS)


def _make_agg(C, edge_split, table_3d):
    cpt = (NCHUNK // 32) if edge_split else (NCHUNK // NS)
    SB = 8 if edge_split else 16
    NSB = cpt // SB
    D = 4 if edge_split else 5

    def body(table, src2, dst2, zeros, out, srcI, dstI, rows, acc,
             gsem, ssem, isem):
        s = lax.axis_index("s")
        c = lax.axis_index("c")

        def core_half(cc):
            tref = table.at[cc] if table_3d else table

            @pl.when(s < 15)
            def _():
                pltpu.sync_copy(zeros, acc.at[pl.ds(s * STRIPE, STRIPE), :])

            @pl.when(s == 15)
            def _():
                pltpu.sync_copy(zeros.at[pl.ds(0, LAST_STRIPE), :],
                                acc.at[pl.ds(15 * STRIPE, LAST_STRIPE), :])

            plsc.subcore_barrier()
            chunk0 = ((cc * NS + s) * cpt) if edge_split else (s * cpt)

            def load_idx(slot, sb):
                base = chunk0 + sb * SB
                pltpu.async_copy(src2.at[pl.ds(base, SB)], srcI.at[slot], isem)
                pltpu.async_copy(dst2.at[pl.ds(base, SB)], dstI.at[slot], isem)

            def wait_idx(slot):
                pltpu.make_async_copy(
                    src2.at[pl.ds(chunk0, SB)], srcI.at[slot], isem).wait()
                pltpu.make_async_copy(
                    dst2.at[pl.ds(chunk0, SB)], dstI.at[slot], isem).wait()

            def process(slot):
                hg = [None] * SB
                hs = [None] * SB
                for j in range(SB):
                    par = j % D
                    if j >= D:
                        hs[j - D].wait()
                    hg[j] = pltpu.async_copy(
                        tref.at[srcI.at[slot, j]], rows.at[par], gsem)
                    if j >= 1:
                        hg[j - 1].wait()
                        hs[j - 1] = pltpu.async_copy(
                            rows.at[(j - 1) % D], acc.at[dstI.at[slot, j - 1]],
                            ssem, add=True)
                hg[SB - 1].wait()
                hs[SB - 1] = pltpu.async_copy(
                    rows.at[(SB - 1) % D], acc.at[dstI.at[slot, SB - 1]],
                    ssem, add=True)
                for j in range(SB - D, SB):
                    hs[j].wait()

            load_idx(0, 0)

            def obody(o2, carry):
                load_idx(1, 2 * o2 + 1)
                wait_idx(0)
                process(0)
                load_idx(0, 2 * o2 + 2)
                wait_idx(1)
                process(1)
                return carry

            lax.fori_loop(0, NSB // 2, obody, 0)
            wait_idx(0)
            process(0)
            plsc.subcore_barrier()

            @pl.when(s < 15)
            def _():
                pltpu.sync_copy(acc.at[pl.ds(s * STRIPE, STRIPE), :],
                                out.at[cc, pl.ds(s * STRIPE, STRIPE), :])

            @pl.when(s == 15)
            def _():
                pltpu.sync_copy(acc.at[pl.ds(15 * STRIPE, LAST_STRIPE), :],
                                out.at[cc, pl.ds(15 * STRIPE, LAST_STRIPE), :])

        @pl.when(c == 0)
        def _():
            core_half(0)

        @pl.when(c == 1)
        def _():
            core_half(1)

    return pl.kernel(
        body,
        out_type=jax.ShapeDtypeStruct((2, N, C), jnp.float32),
        mesh=_mk_mesh(),
        compiler_params=pltpu.CompilerParams(use_tc_tiling_on_sc=False),
        scratch_types=[
            pltpu.VMEM((2, SB, K), jnp.int32),
            pltpu.VMEM((2, SB, K), jnp.int32),
            pltpu.VMEM((D, K, C), jnp.float32),
            pltpu.VMEM_SHARED((N, C), jnp.float32),
            pltpu.SemaphoreType.DMA,
            pltpu.SemaphoreType.DMA,
            pltpu.SemaphoreType.DMA,
        ],
    )


def _relu(v):
    return jnp.maximum(v, 0.0)


def _tc_layer1(s1_ref, x_ref, wl1, wr1, b1, wl2, wr2, b2, p2_ref, q2_ref, dinv_ref):
    s1 = s1_ref[0] + s1_ref[1]
    invd = 1.0 / jnp.maximum(s1[:, 3:4], 1.0)
    agg = s1[:, 0:3] * invd
    h = _relu(jnp.dot(agg, wl1[...], preferred_element_type=jnp.float32)
              + jnp.dot(x_ref[...], wr1[...], preferred_element_type=jnp.float32)
              + b1[...])
    p = jnp.dot(h, wl2[...], preferred_element_type=jnp.float32)
    p2_ref[0] = p[:, 0:32]
    p2_ref[1] = p[:, 32:64]
    q2_ref[...] = jnp.dot(h, wr2[...], preferred_element_type=jnp.float32) + b2[...]
    dinv_ref[...] = invd


def _tc_mid(t_ref, q_ref, dinv_ref, wl, wr, b, pn_ref, qn_ref):
    t = jnp.concatenate([t_ref[0], t_ref[1]], axis=1)
    h = _relu(t * dinv_ref[...] + q_ref[...])
    p = jnp.dot(h, wl[...], preferred_element_type=jnp.float32)
    pn_ref[0] = p[:, 0:32]
    pn_ref[1] = p[:, 32:64]
    qn_ref[...] = jnp.dot(h, wr[...], preferred_element_type=jnp.float32) + b[...]


def _tc_last(t_ref, q_ref, dinv_ref, batch_ref, fcw, fcb, out_ref, sums_ref):
    i = pl.program_id(0)

    @pl.when(i == 0)
    def _():
        sums_ref[...] = jnp.zeros_like(sums_ref)

    t = jnp.concatenate([t_ref[0], t_ref[1]], axis=1)
    h = _relu(t * dinv_ref[...] + q_ref[...])
    hh = jnp.concatenate([h, jnp.ones((RB, 8), jnp.float32)], axis=1)
    onehot = (batch_ref[...] ==
              lax.broadcasted_iota(jnp.int32, (1, G), 1)).astype(jnp.float32)
    sums_ref[...] += lax.dot_general(
        onehot, hh, (((0,), (0,)), ((), ())),
        preferred_element_type=jnp.float32)

    @pl.when(i == GRID - 1)
    def _():
        sm = sums_ref[...]
        mean = sm[:, 0:64] / jnp.maximum(sm[:, 64:65], 1.0)
        o = jnp.dot(mean, fcw[...], preferred_element_type=jnp.float32) + fcb[...]
        nrm = jnp.sqrt(jnp.sum(o * o, axis=1, keepdims=True))
        out_ref[...] = o / jnp.maximum(nrm, 1e-12)


def _full(shape):
    return pl.BlockSpec(shape, lambda *args: tuple(0 for _ in shape))


def kernel(x, edge_index, batch, Wl1, bl1, Wr1, Wl2, bl2, Wr2, Wl3, bl3, Wr3,
           Wl4, bl4, Wr4, fcW, fcb):
    src2 = edge_index[0].reshape(NCHUNK, K)
    dst2 = edge_index[1].reshape(NCHUNK, K)
    batch_col = batch.reshape(N, 1)
    xpad = jnp.concatenate(
        [x, jnp.ones((N, 1), jnp.float32), jnp.zeros((N, 12), jnp.float32)], axis=1)
    zeros16 = jnp.zeros((STRIPE, 16), jnp.float32)
    zeros32 = jnp.zeros((STRIPE, 32), jnp.float32)

    s1 = _make_agg(16, edge_split=True, table_3d=False)(xpad, src2, dst2, zeros16)

    p2, q2, dinv = pl.pallas_call(
        _tc_layer1,
        grid=(GRID,),
        in_specs=[
            pl.BlockSpec((2, RB, 16), lambda i: (0, i, 0)),
            pl.BlockSpec((RB, 3), lambda i: (i, 0)),
            _full((3, 64)), _full((3, 64)), _full((1, 64)),
            _full((64, 64)), _full((64, 64)), _full((1, 64)),
        ],
        out_specs=[
            pl.BlockSpec((2, RB, 32), lambda i: (0, i, 0)),
            pl.BlockSpec((RB, 64), lambda i: (i, 0)),
            pl.BlockSpec((RB, 1), lambda i: (i, 0)),
        ],
        out_shape=[
            jax.ShapeDtypeStruct((2, N, 32), jnp.float32),
            jax.ShapeDtypeStruct((N, 64), jnp.float32),
            jax.ShapeDtypeStruct((N, 1), jnp.float32),
        ],
    )(s1, x, Wl1, Wr1, bl1.reshape(1, 64), Wl2, Wr2, bl2.reshape(1, 64))

    agg32 = _make_agg(32, edge_split=False, table_3d=True)

    def mid(t, q, wl, wr, b):
        return pl.pallas_call(
            _tc_mid,
            grid=(GRID,),
            in_specs=[
                pl.BlockSpec((2, RB, 32), lambda i: (0, i, 0)),
                pl.BlockSpec((RB, 64), lambda i: (i, 0)),
                pl.BlockSpec((RB, 1), lambda i: (i, 0)),
                _full((64, 64)), _full((64, 64)), _full((1, 64)),
            ],
            out_specs=[
                pl.BlockSpec((2, RB, 32), lambda i: (0, i, 0)),
                pl.BlockSpec((RB, 64), lambda i: (i, 0)),
            ],
            out_shape=[
                jax.ShapeDtypeStruct((2, N, 32), jnp.float32),
                jax.ShapeDtypeStruct((N, 64), jnp.float32),
            ],
        )(t, q, dinv, wl, wr, b.reshape(1, 64))

    t2 = agg32(p2, src2, dst2, zeros32)
    p3, q3 = mid(t2, q2, Wl3, Wr3, bl3)
    t3 = agg32(p3, src2, dst2, zeros32)
    p4, q4 = mid(t3, q3, Wl4, Wr4, bl4)
    t4 = agg32(p4, src2, dst2, zeros32)

    out = pl.pallas_call(
        _tc_last,
        grid=(GRID,),
        in_specs=[
            pl.BlockSpec((2, RB, 32), lambda i: (0, i, 0)),
            pl.BlockSpec((RB, 64), lambda i: (i, 0)),
            pl.BlockSpec((RB, 1), lambda i: (i, 0)),
            pl.BlockSpec((RB, 1), lambda i: (i, 0)),
            _full((64, 128)), _full((1, 128)),
        ],
        out_specs=pl.BlockSpec((G, 128), lambda i: (0, 0)),
        out_shape=jax.ShapeDtypeStruct((G, 128), jnp.float32),
        scratch_shapes=[pltpu.VMEM((G, 72), jnp.float32)],
    )(t4, q4, dinv, batch_col, fcW, fcb.reshape(1, 128))
    return out


# speedup vs baseline: 11.9737x; 1.0006x over previous
import jax
import jax.numpy as jnp
from jax import lax
from jax.experimental import pallas as pl
from jax.experimental.pallas import tpu as pltpu
from jax.experimental.pallas import tpu_sc as plsc

N = 50000
E = 800000
G = 64
K = 125
NCHUNK = E // K
RB = 2000
GRID = N // RB
NS = 16
STRIPE = 3128
LAST_STRIPE = N - 15 * STRIPE


def _mk_mesh():
    return plsc.VectorSubcoreMesh(
        core_axis_name="c", subcore_axis_name="s", num_cores=2, num_subcores=NS)


def _make_agg(C, edge_split, table_3d):
    cpt = (NCHUNK // 32) if edge_split else (NCHUNK // NS)
    SB = 8 if edge_split else 16
    NSB = cpt // SB
    D = 8 if edge_split else 5

    def body(table, src2, dst2, zeros, out, srcI, dstI, rows, acc,
             gsem, ssem, isem):
        s = lax.axis_index("s")
        c = lax.axis_index("c")

        def core_half(cc):
            tref = table.at[cc] if table_3d else table

            @pl.when(s < 15)
            def _():
                pltpu.sync_copy(zeros, acc.at[pl.ds(s * STRIPE, STRIPE), :])

            @pl.when(s == 15)
            def _():
                pltpu.sync_copy(zeros.at[pl.ds(0, LAST_STRIPE), :],
                                acc.at[pl.ds(15 * STRIPE, LAST_STRIPE), :])

            plsc.subcore_barrier()
            chunk0 = ((cc * NS + s) * cpt) if edge_split else (s * cpt)

            def load_idx(slot, sb):
                base = chunk0 + sb * SB
                pltpu.async_copy(src2.at[pl.ds(base, SB)], srcI.at[slot], isem)
                pltpu.async_copy(dst2.at[pl.ds(base, SB)], dstI.at[slot], isem)

            def wait_idx(slot):
                pltpu.make_async_copy(
                    src2.at[pl.ds(chunk0, SB)], srcI.at[slot], isem).wait()
                pltpu.make_async_copy(
                    dst2.at[pl.ds(chunk0, SB)], dstI.at[slot], isem).wait()

            def process(slot):
                hg = [None] * SB
                hs = [None] * SB
                for j in range(SB):
                    par = j % D
                    if j >= D:
                        hs[j - D].wait()
                    hg[j] = pltpu.async_copy(
                        tref.at[srcI.at[slot, j]], rows.at[par], gsem)
                    if j >= 1:
                        hg[j - 1].wait()
                        hs[j - 1] = pltpu.async_copy(
                            rows.at[(j - 1) % D], acc.at[dstI.at[slot, j - 1]],
                            ssem, add=True)
                hg[SB - 1].wait()
                hs[SB - 1] = pltpu.async_copy(
                    rows.at[(SB - 1) % D], acc.at[dstI.at[slot, SB - 1]],
                    ssem, add=True)
                for j in range(SB - D, SB):
                    hs[j].wait()

            load_idx(0, 0)

            def obody(o2, carry):
                load_idx(1, 2 * o2 + 1)
                wait_idx(0)
                process(0)
                load_idx(0, 2 * o2 + 2)
                wait_idx(1)
                process(1)
                return carry

            lax.fori_loop(0, NSB // 2, obody, 0)
            wait_idx(0)
            process(0)
            plsc.subcore_barrier()

            @pl.when(s < 15)
            def _():
                pltpu.sync_copy(acc.at[pl.ds(s * STRIPE, STRIPE), :],
                                out.at[cc, pl.ds(s * STRIPE, STRIPE), :])

            @pl.when(s == 15)
            def _():
                pltpu.sync_copy(acc.at[pl.ds(15 * STRIPE, LAST_STRIPE), :],
                                out.at[cc, pl.ds(15 * STRIPE, LAST_STRIPE), :])

        @pl.when(c == 0)
        def _():
            core_half(0)

        @pl.when(c == 1)
        def _():
            core_half(1)

    return pl.kernel(
        body,
        out_type=jax.ShapeDtypeStruct((2, N, C), jnp.float32),
        mesh=_mk_mesh(),
        compiler_params=pltpu.CompilerParams(use_tc_tiling_on_sc=False),
        scratch_types=[
            pltpu.VMEM((2, SB, K), jnp.int32),
            pltpu.VMEM((2, SB, K), jnp.int32),
            pltpu.VMEM((D, K, C), jnp.float32),
            pltpu.VMEM_SHARED((N, C), jnp.float32),
            pltpu.SemaphoreType.DMA,
            pltpu.SemaphoreType.DMA,
            pltpu.SemaphoreType.DMA,
        ],
    )


def _relu(v):
    return jnp.maximum(v, 0.0)


def _tc_layer1(s1_ref, x_ref, wl1, wr1, b1, wl2, wr2, b2, p2_ref, q2_ref, dinv_ref):
    s1 = s1_ref[0] + s1_ref[1]
    invd = 1.0 / jnp.maximum(s1[:, 3:4], 1.0)
    agg = s1[:, 0:3] * invd
    h = _relu(jnp.dot(agg, wl1[...], preferred_element_type=jnp.float32)
              + jnp.dot(x_ref[...], wr1[...], preferred_element_type=jnp.float32)
              + b1[...])
    p = jnp.dot(h, wl2[...], preferred_element_type=jnp.float32)
    p2_ref[0] = p[:, 0:32]
    p2_ref[1] = p[:, 32:64]
    q2_ref[...] = jnp.dot(h, wr2[...], preferred_element_type=jnp.float32) + b2[...]
    dinv_ref[...] = invd


def _tc_mid(t_ref, q_ref, dinv_ref, wl, wr, b, pn_ref, qn_ref):
    t = jnp.concatenate([t_ref[0], t_ref[1]], axis=1)
    h = _relu(t * dinv_ref[...] + q_ref[...])
    p = jnp.dot(h, wl[...], preferred_element_type=jnp.float32)
    pn_ref[0] = p[:, 0:32]
    pn_ref[1] = p[:, 32:64]
    qn_ref[...] = jnp.dot(h, wr[...], preferred_element_type=jnp.float32) + b[...]


def _tc_last(t_ref, q_ref, dinv_ref, batch_ref, fcw, fcb, out_ref, sums_ref):
    i = pl.program_id(0)

    @pl.when(i == 0)
    def _():
        sums_ref[...] = jnp.zeros_like(sums_ref)

    t = jnp.concatenate([t_ref[0], t_ref[1]], axis=1)
    h = _relu(t * dinv_ref[...] + q_ref[...])
    hh = jnp.concatenate([h, jnp.ones((RB, 8), jnp.float32)], axis=1)
    onehot = (batch_ref[...] ==
              lax.broadcasted_iota(jnp.int32, (1, G), 1)).astype(jnp.float32)
    sums_ref[...] += lax.dot_general(
        onehot, hh, (((0,), (0,)), ((), ())),
        preferred_element_type=jnp.float32)

    @pl.when(i == GRID - 1)
    def _():
        sm = sums_ref[...]
        mean = sm[:, 0:64] / jnp.maximum(sm[:, 64:65], 1.0)
        o = jnp.dot(mean, fcw[...], preferred_element_type=jnp.float32) + fcb[...]
        nrm = jnp.sqrt(jnp.sum(o * o, axis=1, keepdims=True))
        out_ref[...] = o / jnp.maximum(nrm, 1e-12)


def _full(shape):
    return pl.BlockSpec(shape, lambda *args: tuple(0 for _ in shape))


def kernel(x, edge_index, batch, Wl1, bl1, Wr1, Wl2, bl2, Wr2, Wl3, bl3, Wr3,
           Wl4, bl4, Wr4, fcW, fcb):
    src2 = edge_index[0].reshape(NCHUNK, K)
    dst2 = edge_index[1].reshape(NCHUNK, K)
    batch_col = batch.reshape(N, 1)
    xpad = jnp.concatenate(
        [x, jnp.ones((N, 1), jnp.float32), jnp.zeros((N, 12), jnp.float32)], axis=1)
    zeros16 = jnp.zeros((STRIPE, 16), jnp.float32)
    zeros32 = jnp.zeros((STRIPE, 32), jnp.float32)

    s1 = _make_agg(16, edge_split=True, table_3d=False)(xpad, src2, dst2, zeros16)

    p2, q2, dinv = pl.pallas_call(
        _tc_layer1,
        grid=(GRID,),
        in_specs=[
            pl.BlockSpec((2, RB, 16), lambda i: (0, i, 0)),
            pl.BlockSpec((RB, 3), lambda i: (i, 0)),
            _full((3, 64)), _full((3, 64)), _full((1, 64)),
            _full((64, 64)), _full((64, 64)), _full((1, 64)),
        ],
        out_specs=[
            pl.BlockSpec((2, RB, 32), lambda i: (0, i, 0)),
            pl.BlockSpec((RB, 64), lambda i: (i, 0)),
            pl.BlockSpec((RB, 1), lambda i: (i, 0)),
        ],
        out_shape=[
            jax.ShapeDtypeStruct((2, N, 32), jnp.float32),
            jax.ShapeDtypeStruct((N, 64), jnp.float32),
            jax.ShapeDtypeStruct((N, 1), jnp.float32),
        ],
    )(s1, x, Wl1, Wr1, bl1.reshape(1, 64), Wl2, Wr2, bl2.reshape(1, 64))

    agg32 = _make_agg(32, edge_split=False, table_3d=True)

    def mid(t, q, wl, wr, b):
        return pl.pallas_call(
            _tc_mid,
            grid=(GRID,),
            in_specs=[
                pl.BlockSpec((2, RB, 32), lambda i: (0, i, 0)),
                pl.BlockSpec((RB, 64), lambda i: (i, 0)),
                pl.BlockSpec((RB, 1), lambda i: (i, 0)),
                _full((64, 64)), _full((64, 64)), _full((1, 64)),
            ],
            out_specs=[
                pl.BlockSpec((2, RB, 32), lambda i: (0, i, 0)),
                pl.BlockSpec((RB, 64), lambda i: (i, 0)),
            ],
            out_shape=[
                jax.ShapeDtypeStruct((2, N, 32), jnp.float32),
                jax.ShapeDtypeStruct((N, 64), jnp.float32),
            ],
        )(t, q, dinv, wl, wr, b.reshape(1, 64))

    t2 = agg32(p2, src2, dst2, zeros32)
    p3, q3 = mid(t2, q2, Wl3, Wr3, bl3)
    t3 = agg32(p3, src2, dst2, zeros32)
    p4, q4 = mid(t3, q3, Wl4, Wr4, bl4)
    t4 = agg32(p4, src2, dst2, zeros32)

    out = pl.pallas_call(
        _tc_last,
        grid=(GRID,),
        in_specs=[
            pl.BlockSpec((2, RB, 32), lambda i: (0, i, 0)),
            pl.BlockSpec((RB, 64), lambda i: (i, 0)),
            pl.BlockSpec((RB, 1), lambda i: (i, 0)),
            pl.BlockSpec((RB, 1), lambda i: (i, 0)),
            _full((64, 128)), _full((1, 128)),
        ],
        out_specs=pl.BlockSpec((G, 128), lambda i: (0, 0)),
        out_shape=jax.ShapeDtypeStruct((G, 128), jnp.float32),
        scratch_shapes=[pltpu.VMEM((G, 72), jnp.float32)],
    )(t4, q4, dinv, batch_col, fcW, fcb.reshape(1, 128))
    return out


# speedup vs baseline: 12.1033x; 1.0108x over previous
import jax
import jax.numpy as jnp
from jax import lax
from jax.experimental import pallas as pl
from jax.experimental.pallas import tpu as pltpu
from jax.experimental.pallas import tpu_sc as plsc

N = 50000
E = 800000
G = 64
K = 125
NCHUNK = E // K
RB = 5000
GRID = N // RB
NS = 16
STRIPE = 3128
LAST_STRIPE = N - 15 * STRIPE


def _mk_mesh():
    return plsc.VectorSubcoreMesh(
        core_axis_name="c", subcore_axis_name="s", num_cores=2, num_subcores=NS)


def _make_agg(C, edge_split, table_3d):
    cpt = (NCHUNK // 32) if edge_split else (NCHUNK // NS)
    SB = 8 if edge_split else 16
    NSB = cpt // SB
    D = 8 if edge_split else 5

    def body(table, src2, dst2, zeros, out, srcI, dstI, rows, acc,
             gsem, ssem, isem):
        s = lax.axis_index("s")
        c = lax.axis_index("c")

        def core_half(cc):
            tref = table.at[cc] if table_3d else table

            @pl.when(s < 15)
            def _():
                pltpu.sync_copy(zeros, acc.at[pl.ds(s * STRIPE, STRIPE), :])

            @pl.when(s == 15)
            def _():
                pltpu.sync_copy(zeros.at[pl.ds(0, LAST_STRIPE), :],
                                acc.at[pl.ds(15 * STRIPE, LAST_STRIPE), :])

            plsc.subcore_barrier()
            chunk0 = ((cc * NS + s) * cpt) if edge_split else (s * cpt)

            def load_idx(slot, sb):
                base = chunk0 + sb * SB
                pltpu.async_copy(src2.at[pl.ds(base, SB)], srcI.at[slot], isem)
                pltpu.async_copy(dst2.at[pl.ds(base, SB)], dstI.at[slot], isem)

            def wait_idx(slot):
                pltpu.make_async_copy(
                    src2.at[pl.ds(chunk0, SB)], srcI.at[slot], isem).wait()
                pltpu.make_async_copy(
                    dst2.at[pl.ds(chunk0, SB)], dstI.at[slot], isem).wait()

            def process(slot):
                hg = [None] * SB
                hs = [None] * SB
                for j in range(SB):
                    par = j % D
                    if j >= D:
                        hs[j - D].wait()
                    hg[j] = pltpu.async_copy(
                        tref.at[srcI.at[slot, j]], rows.at[par], gsem)
                    if j >= 1:
                        hg[j - 1].wait()
                        hs[j - 1] = pltpu.async_copy(
                            rows.at[(j - 1) % D], acc.at[dstI.at[slot, j - 1]],
                            ssem, add=True)
                hg[SB - 1].wait()
                hs[SB - 1] = pltpu.async_copy(
                    rows.at[(SB - 1) % D], acc.at[dstI.at[slot, SB - 1]],
                    ssem, add=True)
                for j in range(SB - D, SB):
                    hs[j].wait()

            load_idx(0, 0)

            def obody(o2, carry):
                load_idx(1, 2 * o2 + 1)
                wait_idx(0)
                process(0)
                load_idx(0, 2 * o2 + 2)
                wait_idx(1)
                process(1)
                return carry

            lax.fori_loop(0, NSB // 2, obody, 0)
            wait_idx(0)
            process(0)
            plsc.subcore_barrier()

            @pl.when(s < 15)
            def _():
                pltpu.sync_copy(acc.at[pl.ds(s * STRIPE, STRIPE), :],
                                out.at[cc, pl.ds(s * STRIPE, STRIPE), :])

            @pl.when(s == 15)
            def _():
                pltpu.sync_copy(acc.at[pl.ds(15 * STRIPE, LAST_STRIPE), :],
                                out.at[cc, pl.ds(15 * STRIPE, LAST_STRIPE), :])

        @pl.when(c == 0)
        def _():
            core_half(0)

        @pl.when(c == 1)
        def _():
            core_half(1)

    return pl.kernel(
        body,
        out_type=jax.ShapeDtypeStruct((2, N, C), jnp.float32),
        mesh=_mk_mesh(),
        compiler_params=pltpu.CompilerParams(use_tc_tiling_on_sc=False),
        scratch_types=[
            pltpu.VMEM((2, SB, K), jnp.int32),
            pltpu.VMEM((2, SB, K), jnp.int32),
            pltpu.VMEM((D, K, C), jnp.float32),
            pltpu.VMEM_SHARED((N, C), jnp.float32),
            pltpu.SemaphoreType.DMA,
            pltpu.SemaphoreType.DMA,
            pltpu.SemaphoreType.DMA,
        ],
    )


def _relu(v):
    return jnp.maximum(v, 0.0)


def _tc_layer1(s1_ref, x_ref, wl1, wr1, b1, wl2, wr2, b2, p2_ref, q2_ref, dinv_ref):
    s1 = s1_ref[0] + s1_ref[1]
    invd = 1.0 / jnp.maximum(s1[:, 3:4], 1.0)
    agg = s1[:, 0:3] * invd
    h = _relu(jnp.dot(agg, wl1[...], preferred_element_type=jnp.float32)
              + jnp.dot(x_ref[...], wr1[...], preferred_element_type=jnp.float32)
              + b1[...])
    p = jnp.dot(h, wl2[...], preferred_element_type=jnp.float32)
    p2_ref[0] = p[:, 0:32]
    p2_ref[1] = p[:, 32:64]
    q2_ref[...] = jnp.dot(h, wr2[...], preferred_element_type=jnp.float32) + b2[...]
    dinv_ref[...] = invd


def _tc_mid(t_ref, q_ref, dinv_ref, wl, wr, b, pn_ref, qn_ref):
    t = jnp.concatenate([t_ref[0], t_ref[1]], axis=1)
    h = _relu(t * dinv_ref[...] + q_ref[...])
    p = jnp.dot(h, wl[...], preferred_element_type=jnp.float32)
    pn_ref[0] = p[:, 0:32]
    pn_ref[1] = p[:, 32:64]
    qn_ref[...] = jnp.dot(h, wr[...], preferred_element_type=jnp.float32) + b[...]


def _tc_last(t_ref, q_ref, dinv_ref, batch_ref, fcw, fcb, out_ref, sums_ref):
    i = pl.program_id(0)

    @pl.when(i == 0)
    def _():
        sums_ref[...] = jnp.zeros_like(sums_ref)

    t = jnp.concatenate([t_ref[0], t_ref[1]], axis=1)
    h = _relu(t * dinv_ref[...] + q_ref[...])
    hh = jnp.concatenate([h, jnp.ones((RB, 8), jnp.float32)], axis=1)
    onehot = (batch_ref[...] ==
              lax.broadcasted_iota(jnp.int32, (1, G), 1)).astype(jnp.float32)
    sums_ref[...] += lax.dot_general(
        onehot, hh, (((0,), (0,)), ((), ())),
        preferred_element_type=jnp.float32)

    @pl.when(i == GRID - 1)
    def _():
        sm = sums_ref[...]
        mean = sm[:, 0:64] / jnp.maximum(sm[:, 64:65], 1.0)
        o = jnp.dot(mean, fcw[...], preferred_element_type=jnp.float32) + fcb[...]
        nrm = jnp.sqrt(jnp.sum(o * o, axis=1, keepdims=True))
        out_ref[...] = o / jnp.maximum(nrm, 1e-12)


def _full(shape):
    return pl.BlockSpec(shape, lambda *args: tuple(0 for _ in shape))


def kernel(x, edge_index, batch, Wl1, bl1, Wr1, Wl2, bl2, Wr2, Wl3, bl3, Wr3,
           Wl4, bl4, Wr4, fcW, fcb):
    src2 = edge_index[0].reshape(NCHUNK, K)
    dst2 = edge_index[1].reshape(NCHUNK, K)
    batch_col = batch.reshape(N, 1)
    xpad = jnp.concatenate(
        [x, jnp.ones((N, 1), jnp.float32), jnp.zeros((N, 12), jnp.float32)], axis=1)
    zeros16 = jnp.zeros((STRIPE, 16), jnp.float32)
    zeros32 = jnp.zeros((STRIPE, 32), jnp.float32)

    s1 = _make_agg(16, edge_split=True, table_3d=False)(xpad, src2, dst2, zeros16)

    p2, q2, dinv = pl.pallas_call(
        _tc_layer1,
        grid=(GRID,),
        in_specs=[
            pl.BlockSpec((2, RB, 16), lambda i: (0, i, 0)),
            pl.BlockSpec((RB, 3), lambda i: (i, 0)),
            _full((3, 64)), _full((3, 64)), _full((1, 64)),
            _full((64, 64)), _full((64, 64)), _full((1, 64)),
        ],
        out_specs=[
            pl.BlockSpec((2, RB, 32), lambda i: (0, i, 0)),
            pl.BlockSpec((RB, 64), lambda i: (i, 0)),
            pl.BlockSpec((RB, 1), lambda i: (i, 0)),
        ],
        out_shape=[
            jax.ShapeDtypeStruct((2, N, 32), jnp.float32),
            jax.ShapeDtypeStruct((N, 64), jnp.float32),
            jax.ShapeDtypeStruct((N, 1), jnp.float32),
        ],
    )(s1, x, Wl1, Wr1, bl1.reshape(1, 64), Wl2, Wr2, bl2.reshape(1, 64))

    agg32 = _make_agg(32, edge_split=False, table_3d=True)

    def mid(t, q, wl, wr, b):
        return pl.pallas_call(
            _tc_mid,
            grid=(GRID,),
            in_specs=[
                pl.BlockSpec((2, RB, 32), lambda i: (0, i, 0)),
                pl.BlockSpec((RB, 64), lambda i: (i, 0)),
                pl.BlockSpec((RB, 1), lambda i: (i, 0)),
                _full((64, 64)), _full((64, 64)), _full((1, 64)),
            ],
            out_specs=[
                pl.BlockSpec((2, RB, 32), lambda i: (0, i, 0)),
                pl.BlockSpec((RB, 64), lambda i: (i, 0)),
            ],
            out_shape=[
                jax.ShapeDtypeStruct((2, N, 32), jnp.float32),
                jax.ShapeDtypeStruct((N, 64), jnp.float32),
            ],
        )(t, q, dinv, wl, wr, b.reshape(1, 64))

    t2 = agg32(p2, src2, dst2, zeros32)
    p3, q3 = mid(t2, q2, Wl3, Wr3, bl3)
    t3 = agg32(p3, src2, dst2, zeros32)
    p4, q4 = mid(t3, q3, Wl4, Wr4, bl4)
    t4 = agg32(p4, src2, dst2, zeros32)

    out = pl.pallas_call(
        _tc_last,
        grid=(GRID,),
        in_specs=[
            pl.BlockSpec((2, RB, 32), lambda i: (0, i, 0)),
            pl.BlockSpec((RB, 64), lambda i: (i, 0)),
            pl.BlockSpec((RB, 1), lambda i: (i, 0)),
            pl.BlockSpec((RB, 1), lambda i: (i, 0)),
            _full((64, 128)), _full((1, 128)),
        ],
        out_specs=pl.BlockSpec((G, 128), lambda i: (0, 0)),
        out_shape=jax.ShapeDtypeStruct((G, 128), jnp.float32),
        scratch_shapes=[pltpu.VMEM((G, 72), jnp.float32)],
    )(t4, q4, dinv, batch_col, fcW, fcb.reshape(1, 128))
    return out
